# baseline (speedup 1.0000x reference)
import functools

import jax
import jax.numpy as jnp
from jax import lax
from jax.experimental import pallas as pl
from jax.experimental.pallas import tpu as pltpu
from jax.experimental.pallas import tpu_sc as plsc

B, S, D = 4, 8192, 1024
NC, NS = 2, 16
NW = NC * NS
ROWS_PER_W = S // NW
CH_ROWS = 32
CH = CH_ROWS * D
N_CHUNKS = ROWS_PER_W // CH_ROWS


def _sc_body(x_hbm, e_hbm, o_hbm, xbuf, ebuf):
    wid = lax.axis_index("s") * NC + lax.axis_index("c")
    w_base = wid * (ROWS_PER_W * D)
    for c in range(N_CHUNKS):
        e_off = w_base + c * CH
        pltpu.sync_copy(e_hbm.at[pl.ds(e_off, CH)], ebuf)
        for b in range(B):
            x_off = b * (S * D) + e_off
            pltpu.sync_copy(x_hbm.at[pl.ds(x_off, CH)], xbuf)

            def add4(i, _):
                for k in range(4):
                    sl = pl.ds(i * 64 + k * 16, 16)
                    xbuf[sl] = xbuf[sl] + ebuf[sl]
                return 0

            lax.fori_loop(0, CH // 64, add4, 0)
            pltpu.sync_copy(xbuf, o_hbm.at[pl.ds(x_off, CH)])


def _sc_add(x1, e1):
    mesh = plsc.VectorSubcoreMesh(core_axis_name="c", subcore_axis_name="s")
    k = functools.partial(
        pl.kernel,
        mesh=mesh,
        out_type=jax.ShapeDtypeStruct((B * S * D,), jnp.float32),
        scratch_types=[
            pltpu.VMEM((CH,), jnp.float32),
            pltpu.VMEM((CH,), jnp.float32),
        ],
    )(_sc_body)
    return k(x1, e1)


def kernel(inputs, embeddings):
    x1 = inputs.reshape(-1)
    e1 = embeddings[:S].reshape(-1)
    out = _sc_add(x1, e1)
    return out.reshape(B, S, D)


# speedup vs baseline: 4.9626x; 4.9626x over previous
import jax
import jax.numpy as jnp
from jax.experimental import pallas as pl


def _add_kernel(x_ref, e_ref, o_ref):
    o_ref[...] = x_ref[...] + e_ref[...][None, :, :]


def kernel(inputs, embeddings):
    B, S, D = inputs.shape
    BS = 1024
    return pl.pallas_call(
        _add_kernel,
        grid=(S // BS, B),
        in_specs=[
            pl.BlockSpec((1, BS, D), lambda i, b: (b, i, 0)),
            pl.BlockSpec((BS, D), lambda i, b: (i, 0)),
        ],
        out_specs=pl.BlockSpec((1, BS, D), lambda i, b: (b, i, 0)),
        out_shape=jax.ShapeDtypeStruct((B, S, D), inputs.dtype),
    )(inputs, embeddings[:S])


# speedup vs baseline: 5.1777x; 1.0433x over previous
import jax
import jax.numpy as jnp
from jax.experimental import pallas as pl


def _add_kernel(x_ref, e_ref, o_ref):
    o_ref[...] = x_ref[...] + e_ref[...][None, :, :]


def kernel(inputs, embeddings):
    B, S, D = inputs.shape
    BS = 2048
    return pl.pallas_call(
        _add_kernel,
        grid=(S // BS, B),
        in_specs=[
            pl.BlockSpec((1, BS, D), lambda i, b: (b, i, 0)),
            pl.BlockSpec((BS, D), lambda i, b: (i, 0)),
        ],
        out_specs=pl.BlockSpec((1, BS, D), lambda i, b: (b, i, 0)),
        out_shape=jax.ShapeDtypeStruct((B, S, D), inputs.dtype),
    )(inputs, embeddings[:S])
